# baseline (speedup 1.0000x reference)
import jax
import jax.numpy as jnp
from jax.experimental import pallas as pl
from jax.experimental.pallas import tpu as pltpu

_B, _H, _W, _D = 64, 24, 24, 768
_K = 9
_BB = 4


def _build_wall(dw_ref, pw_ref, wall_ref):
    rio = jax.lax.broadcasted_iota(jnp.int32, (_K, 128), 0)
    cio = jax.lax.broadcasted_iota(jnp.int32, (_K, 128), 1)
    live = cio < 81
    emat = ((cio // _K == rio) & live).astype(jnp.float32)
    fmat = ((cio % _K == rio) & live).astype(jnp.float32)
    d128 = jnp.dot(dw_ref[...], emat, preferred_element_type=jnp.float32)
    p128 = jax.lax.dot_general(
        pw_ref[...], fmat, (((0,), (0,)), ((), ())),
        preferred_element_type=jnp.float32)
    wall_ref[...] = d128 * p128


def _conv_head_body(x_ref, dw_ref, pw_ref, b_ref, o_ref, wall_ref):
    @pl.when(pl.program_id(0) == 0)
    def _():
        _build_wall(dw_ref, pw_ref, wall_ref)

    x = x_ref[...].reshape(_BB * _H * _W, _D)
    z = jnp.dot(x, wall_ref[...], preferred_element_type=jnp.float32)
    z = z.reshape(_BB, _H, _W, 128)
    shape = (_BB, _H, _W, 128)
    wio = jax.lax.broadcasted_iota(jnp.int32, shape, 2)
    cio = jax.lax.broadcasted_iota(jnp.int32, shape, 3)
    jg = (cio // _K) % 3
    pm = pltpu.roll(z, 1, axis=2)
    pp = pltpu.roll(z, _W - 1, axis=2)
    q = jnp.where(
        jg == 1, z,
        jnp.where((jg == 0) & (wio != 0), pm,
                  jnp.where((jg == 2) & (wio != _W - 1), pp, 0.0)))
    zrow = jnp.zeros((_BB, 1, _W, 128), jnp.float32)
    qp = jnp.concatenate([zrow, q, zrow], axis=1)
    ig = (cio // _K) // 3
    sl = [jax.lax.slice(qp, (0, i, 0, 0), (_BB, i + _H, _W, 128))
          for i in range(3)]
    s = jnp.where(ig == 0, sl[0], jnp.where(ig == 1, sl[1], sl[2]))
    s = s.reshape(_BB * _H * _W, 128)
    tmat = (jax.lax.broadcasted_iota(jnp.int32, (128, 128), 0) % _K
            == jax.lax.broadcasted_iota(jnp.int32, (128, 128), 1))
    tmat &= jax.lax.broadcasted_iota(jnp.int32, (128, 128), 0) < 81
    acc = jnp.dot(s, tmat.astype(jnp.float32),
                  preferred_element_type=jnp.float32)
    logits = jax.lax.slice(acc, (0, 0), (_BB * _H * _W, _K)) + b_ref[0:1, :]
    m = jnp.max(logits, axis=-1, keepdims=True)
    e = jnp.exp(logits - m)
    den = jnp.sum(e, axis=-1, keepdims=True)
    o_ref[...] = (e / den).reshape(_BB, _H, _W, _K)


def kernel(tok2d, dw_w, pw_w, pw_b):
    dwt = dw_w.reshape(_D, 9)
    pwm = pw_w.reshape(_K, _D)
    bias = pw_b.reshape(1, _K)
    out = pl.pallas_call(
        _conv_head_body,
        grid=(_B // _BB,),
        in_specs=[
            pl.BlockSpec((_BB, _H, _W, _D), lambda b: (b, 0, 0, 0)),
            pl.BlockSpec((_D, _K), lambda b: (0, 0)),
            pl.BlockSpec((_K, _D), lambda b: (0, 0)),
            pl.BlockSpec((1, _K), lambda b: (0, 0)),
        ],
        out_specs=pl.BlockSpec((_BB, _H, _W, _K), lambda b: (b, 0, 0, 0)),
        out_shape=jax.ShapeDtypeStruct((_B, _H, _W, _K), jnp.float32),
        scratch_shapes=[pltpu.VMEM((_D, 128), jnp.float32)],
    )(tok2d, dwt, pwm, bias)
    return out


# speedup vs baseline: 1.0412x; 1.0412x over previous
import jax
import jax.numpy as jnp
from jax.experimental import pallas as pl
from jax.experimental.pallas import tpu as pltpu

_B, _H, _W, _D = 64, 24, 24, 768
_K = 9
_BB = 8


def _build_wall(dw_ref, pw_ref, wall_ref):
    rio = jax.lax.broadcasted_iota(jnp.int32, (_K, 128), 0)
    cio = jax.lax.broadcasted_iota(jnp.int32, (_K, 128), 1)
    live = cio < 81
    emat = ((cio // _K == rio) & live).astype(jnp.float32)
    fmat = ((cio % _K == rio) & live).astype(jnp.float32)
    d128 = jnp.dot(dw_ref[...], emat, preferred_element_type=jnp.float32)
    p128 = jax.lax.dot_general(
        pw_ref[...], fmat, (((0,), (0,)), ((), ())),
        preferred_element_type=jnp.float32)
    wall_ref[...] = d128 * p128


def _conv_head_body(x_ref, dw_ref, pw_ref, b_ref, o_ref, wall_ref):
    @pl.when(pl.program_id(0) == 0)
    def _():
        _build_wall(dw_ref, pw_ref, wall_ref)

    x = x_ref[...].reshape(_BB * _H * _W, _D)
    z = jnp.dot(x, wall_ref[...], preferred_element_type=jnp.float32)
    z = z.reshape(_BB, _H, _W, 128)
    shape = (_BB, _H, _W, 128)
    wio = jax.lax.broadcasted_iota(jnp.int32, shape, 2)
    cio = jax.lax.broadcasted_iota(jnp.int32, shape, 3)
    jg = (cio // _K) % 3
    pm = pltpu.roll(z, 1, axis=2)
    pp = pltpu.roll(z, _W - 1, axis=2)
    q = jnp.where(
        jg == 1, z,
        jnp.where((jg == 0) & (wio != 0), pm,
                  jnp.where((jg == 2) & (wio != _W - 1), pp, 0.0)))
    zrow = jnp.zeros((_BB, 1, _W, 128), jnp.float32)
    qp = jnp.concatenate([zrow, q, zrow], axis=1)
    ig = (cio // _K) // 3
    sl = [jax.lax.slice(qp, (0, i, 0, 0), (_BB, i + _H, _W, 128))
          for i in range(3)]
    s = jnp.where(ig == 0, sl[0], jnp.where(ig == 1, sl[1], sl[2]))
    s = s.reshape(_BB * _H * _W, 128)
    tmat = (jax.lax.broadcasted_iota(jnp.int32, (128, 128), 0) % _K
            == jax.lax.broadcasted_iota(jnp.int32, (128, 128), 1))
    tmat &= jax.lax.broadcasted_iota(jnp.int32, (128, 128), 0) < 81
    acc = jnp.dot(s, tmat.astype(jnp.float32),
                  preferred_element_type=jnp.float32)
    logits = jax.lax.slice(acc, (0, 0), (_BB * _H * _W, _K)) + b_ref[0:1, :]
    m = jnp.max(logits, axis=-1, keepdims=True)
    e = jnp.exp(logits - m)
    den = jnp.sum(e, axis=-1, keepdims=True)
    o_ref[...] = (e / den).reshape(_BB, _H, _W, _K)


def kernel(tok2d, dw_w, pw_w, pw_b):
    dwt = dw_w.reshape(_D, 9)
    pwm = pw_w.reshape(_K, _D)
    bias = pw_b.reshape(1, _K)
    out = pl.pallas_call(
        _conv_head_body,
        grid=(_B // _BB,),
        in_specs=[
            pl.BlockSpec((_BB, _H, _W, _D), lambda b: (b, 0, 0, 0)),
            pl.BlockSpec((_D, _K), lambda b: (0, 0)),
            pl.BlockSpec((_K, _D), lambda b: (0, 0)),
            pl.BlockSpec((1, _K), lambda b: (0, 0)),
        ],
        out_specs=pl.BlockSpec((_BB, _H, _W, _K), lambda b: (b, 0, 0, 0)),
        out_shape=jax.ShapeDtypeStruct((_B, _H, _W, _K), jnp.float32),
        scratch_shapes=[pltpu.VMEM((_D, 128), jnp.float32)],
    )(tok2d, dwt, pwm, bias)
    return out
